# Initial kernel scaffold; baseline (speedup 1.0000x reference)
import jax
import jax.numpy as jnp
from jax.experimental import pallas as pl


def kernel(features, target_fearures_0, source_labels, target_labels, queue, queue_labels):
    raise NotImplementedError("write your pallas kernel here")


# speedup vs baseline: 5.1831x; 5.1831x over previous
import functools

import jax
import jax.numpy as jnp
from jax import lax
from jax.experimental import pallas as pl
from jax.experimental.pallas import tpu as pltpu
from jax.experimental.pallas import tpu_sc as plsc

DIM = 512
KQ = 48000
TEMP = 0.007
TOPN = 32
RK = 4
BSRC = 64
BTGT = 256
CHUNK = 128
KP = 49152
NB = 16
LBLK = KP // NB
CPB = LBLK // CHUNK
NCH = KP // CHUNK
BIG = 1 << 30


def _sim_body(feats_ref, tf0_ref, q_ref, sim_ref, sim0_ref, cm_ref, logz_ref,
              tn_ref, m_ref, s_ref):
    k = pl.program_id(0)

    @pl.when(k == 0)
    def _init():
        t = feats_ref[BSRC:, :]
        t0 = tf0_ref[...]
        tn_ref[0:BTGT, :] = t / (jnp.sqrt(jnp.sum(t * t, axis=1, keepdims=True)) + 1e-12)
        tn_ref[BTGT:, :] = t0 / (jnp.sqrt(jnp.sum(t0 * t0, axis=1, keepdims=True)) + 1e-12)
        m_ref[...] = jnp.full((BTGT, 1), -jnp.inf, jnp.float32)
        s_ref[...] = jnp.zeros((BTGT, 1), jnp.float32)

    qblk = q_ref[...]
    src = feats_ref[0:BSRC, :]
    head = jnp.where(k == 0, src, qblk[0:BSRC, :])
    q_eff = jnp.concatenate([head, qblk[BSRC:, :]], axis=0)
    qn = q_eff / (jnp.sqrt(jnp.sum(q_eff * q_eff, axis=1, keepdims=True)) + 1e-12)
    s_all = lax.dot_general(tn_ref[...], qn, (((1,), (1,)), ((), ())),
                            preferred_element_type=jnp.float32)
    sim_b = s_all[0:BTGT, :]
    sim0_b = s_all[BTGT:, :]
    sim_ref[...] = sim_b.reshape(BTGT, CPB, CHUNK)
    sim0_ref[...] = sim0_b.reshape(BTGT, CPB, CHUNK)

    gcol = lax.broadcasted_iota(jnp.int32, (BTGT, LBLK), 1) + k * LBLK
    masked = jnp.where(gcol < KQ, sim_b, -jnp.inf)
    cm_ref[0, :, :] = jnp.max(masked.reshape(BTGT, CPB, CHUNK), axis=2)

    logits = masked / TEMP
    bm = jnp.max(logits, axis=1, keepdims=True)
    new_m = jnp.maximum(m_ref[...], bm)
    s_ref[...] = s_ref[...] * jnp.exp(m_ref[...] - new_m) + \
        jnp.sum(jnp.exp(logits - new_m), axis=1, keepdims=True)
    m_ref[...] = new_m

    @pl.when(k == NB - 1)
    def _fin():
        logz_ref[...] = m_ref[...] + jnp.log(s_ref[...])


def _sel_body(cm_ref, fidx_ref, cidx_ref):
    cm0 = cm_ref[...]
    col = lax.broadcasted_iota(jnp.int32, (BTGT, NCH), 1)
    row = lax.broadcasted_iota(jnp.int32, (BTGT, 1), 0)
    ncol = lax.broadcasted_iota(jnp.int32, (BTGT, TOPN), 1)

    def step(n, carry):
        cm, acc_f, acc_c = carry
        m = jnp.max(cm, axis=1, keepdims=True)
        eq = cm == m
        idx = jnp.min(jnp.where(eq, col, BIG), axis=1, keepdims=True)
        cm = jnp.where(col == idx, -jnp.inf, cm)
        acc_c = jnp.where(ncol == n, idx, acc_c)
        acc_f = jnp.where(ncol == n, row * NCH + idx, acc_f)
        return cm, acc_f, acc_c

    zero = jnp.zeros((BTGT, TOPN), jnp.int32)
    _, acc_f, acc_c = lax.fori_loop(0, TOPN, step, (cm0, zero, zero))
    fidx_ref[...] = acc_f
    cidx_ref[...] = acc_c


def _gather_sc(sim2d, sim02d, lab2d, fidx, cidx):
    info = plsc.get_sparse_core_info()
    nw = info.num_cores * info.num_subcores
    nr = BTGT * TOPN
    bw = nr // nw
    half = bw // 2
    mesh = plsc.VectorSubcoreMesh(core_axis_name="c", subcore_axis_name="s")

    @functools.partial(
        pl.kernel,
        out_type=[
            jax.ShapeDtypeStruct((nr, CHUNK), jnp.float32),
            jax.ShapeDtypeStruct((nr, CHUNK), jnp.float32),
            jax.ShapeDtypeStruct((nr, CHUNK), jnp.int32),
        ],
        mesh=mesh,
        scratch_types=[
            pltpu.VMEM((half,), jnp.int32),
            pltpu.VMEM((half,), jnp.int32),
            pltpu.VMEM((half,), jnp.int32),
            pltpu.VMEM((half,), jnp.int32),
            pltpu.VMEM((half, CHUNK), jnp.float32),
            pltpu.VMEM((half, CHUNK), jnp.float32),
            pltpu.VMEM((half, CHUNK), jnp.float32),
            pltpu.VMEM((half, CHUNK), jnp.float32),
            pltpu.VMEM((half, CHUNK), jnp.int32),
            pltpu.VMEM((half, CHUNK), jnp.int32),
            pltpu.SemaphoreType.DMA,
            pltpu.SemaphoreType.DMA,
        ],
    )
    def gather_kernel(sim_hbm, sim0_hbm, lab_hbm, fidx_hbm, cidx_hbm,
                      cand_hbm, cand0_hbm, clab_hbm,
                      idx_a, idx_b, cidx_a, cidx_b,
                      buf_sa, buf_sb, buf_0a, buf_0b, buf_la, buf_lb,
                      gsem, wsem):
        wid = lax.axis_index("s") * info.num_cores + lax.axis_index("c")
        base = wid * bw
        pltpu.sync_copy(fidx_hbm.at[pl.ds(base, half)], idx_a)
        pltpu.sync_copy(fidx_hbm.at[pl.ds(base + half, half)], idx_b)
        pltpu.sync_copy(cidx_hbm.at[pl.ds(base, half)], cidx_a)
        pltpu.sync_copy(cidx_hbm.at[pl.ds(base + half, half)], cidx_b)
        gathers = [
            (sim_hbm, idx_a, buf_sa, cand_hbm, base),
            (sim_hbm, idx_b, buf_sb, cand_hbm, base + half),
            (sim0_hbm, idx_a, buf_0a, cand0_hbm, base),
            (sim0_hbm, idx_b, buf_0b, cand0_hbm, base + half),
            (lab_hbm, cidx_a, buf_la, clab_hbm, base),
            (lab_hbm, cidx_b, buf_lb, clab_hbm, base + half),
        ]
        copies = [pltpu.async_copy(tbl.at[idx], buf, gsem)
                  for tbl, idx, buf, _, _ in gathers]
        writes = []
        for cp, (_, _, buf, out, off) in zip(copies, gathers):
            cp.wait()
            writes.append(pltpu.async_copy(buf, out.at[pl.ds(off, half)], wsem))
        for wr in writes:
            wr.wait()

    return gather_kernel(sim2d, sim02d, lab2d, fidx, cidx)


def _loss_body(cand_ref, cand0_ref, clab_ref, gcol_ref, logz_ref, srcl_ref,
               tgt_ref, loss_ref, nc_ref):
    cand = cand_ref[...].reshape(BTGT, TOPN, CHUNK)
    cand0 = cand0_ref[...].reshape(BTGT, TOPN, CHUNK)
    jio = lax.broadcasted_iota(jnp.int32, (BTGT, TOPN, CHUNK), 1)
    lio = lax.broadcasted_iota(jnp.int32, (BTGT, TOPN, CHUNK), 2)
    pos_mat = jio * CHUNK + lio

    def _rowmax(x):
        return jnp.max(jnp.max(x, axis=2, keepdims=True), axis=1, keepdims=True)

    def _rowmin(x):
        return jnp.min(jnp.min(x, axis=2, keepdims=True), axis=1, keepdims=True)

    def mask_step(_, v):
        return jnp.where(v == _rowmax(v), -jnp.inf, v)

    v = lax.fori_loop(0, TOPN - 1, mask_step, cand)
    theta = _rowmax(v)
    rank = jnp.where(cand >= theta, cand + cand0, -jnp.inf)

    logz = logz_ref[...]
    vsum = jnp.zeros((BTGT, 1), jnp.float32)
    g0 = jnp.zeros((BTGT, 1), jnp.int32)
    lab_q = jnp.zeros((BTGT, 1), jnp.int32)

    def _rowsum(x):
        return lax.squeeze(
            jnp.sum(jnp.sum(x, axis=2, keepdims=True), axis=1, keepdims=True),
            (2,))

    for n in range(RK):
        m = _rowmax(rank)
        eq = rank == m
        pos = _rowmin(jnp.where(eq, pos_mat, BIG))
        sel = pos_mat == pos
        val = _rowsum(jnp.where(sel, cand, 0.0))
        vsum = vsum + (val / TEMP - logz)
        if n == 0:
            g0 = _rowsum(jnp.where(sel, gcol_ref[...].reshape(BTGT, TOPN, CHUNK), 0))
            lab_q = _rowsum(jnp.where(sel, clab_ref[...].reshape(BTGT, TOPN, CHUNK), 0))
        rank = jnp.where(sel, -jnp.inf, rank)

    loss_ref[...] = -(jnp.sum(vsum, axis=0, keepdims=True) / (BTGT * RK))

    iota64 = lax.broadcasted_iota(jnp.int32, (BTGT, BSRC), 1)
    onehot = jnp.where(iota64 == g0, 1.0, 0.0)
    src_val = lax.dot_general(onehot, srcl_ref[...], (((1,), (0,)), ((), ())),
                              preferred_element_type=jnp.float32)
    pred = jnp.where(g0 < BSRC, src_val.astype(jnp.int32), lab_q)
    nc_ref[...] = jnp.sum(jnp.where(pred == tgt_ref[...], 1, 0),
                          axis=0, keepdims=True)


def kernel(features, target_fearures_0, source_labels, target_labels, queue,
           queue_labels):
    f32 = jnp.float32
    queue_p = jnp.concatenate(
        [queue, jnp.zeros((KP - KQ, DIM), f32)], axis=0)
    qlab_p = jnp.concatenate(
        [queue_labels, jnp.zeros((KP - KQ,), queue_labels.dtype)], axis=0)

    sim3, sim03, cm3, logz = pl.pallas_call(
        _sim_body,
        grid=(NB,),
        in_specs=[
            pl.BlockSpec((BSRC + BTGT, DIM), lambda k: (0, 0)),
            pl.BlockSpec((BTGT, DIM), lambda k: (0, 0)),
            pl.BlockSpec((LBLK, DIM), lambda k: (k, 0)),
        ],
        out_specs=[
            pl.BlockSpec((BTGT, CPB, CHUNK), lambda k: (0, k, 0)),
            pl.BlockSpec((BTGT, CPB, CHUNK), lambda k: (0, k, 0)),
            pl.BlockSpec((1, BTGT, CPB), lambda k: (k, 0, 0)),
            pl.BlockSpec((BTGT, 1), lambda k: (0, 0)),
        ],
        out_shape=[
            jax.ShapeDtypeStruct((BTGT, NCH, CHUNK), f32),
            jax.ShapeDtypeStruct((BTGT, NCH, CHUNK), f32),
            jax.ShapeDtypeStruct((NB, BTGT, CPB), f32),
            jax.ShapeDtypeStruct((BTGT, 1), f32),
        ],
        scratch_shapes=[
            pltpu.VMEM((BTGT + BTGT, DIM), f32),
            pltpu.VMEM((BTGT, 1), f32),
            pltpu.VMEM((BTGT, 1), f32),
        ],
    )(features, target_fearures_0, queue_p)

    cm = cm3.transpose(1, 0, 2).reshape(BTGT, NCH)

    fidx, cidx = pl.pallas_call(
        _sel_body,
        out_shape=[
            jax.ShapeDtypeStruct((BTGT, TOPN), jnp.int32),
            jax.ShapeDtypeStruct((BTGT, TOPN), jnp.int32),
        ],
    )(cm)

    cand, cand0, clab = _gather_sc(
        sim3.reshape(BTGT * NCH, CHUNK),
        sim03.reshape(BTGT * NCH, CHUNK),
        qlab_p.reshape(NCH, CHUNK),
        fidx.reshape(BTGT * TOPN),
        cidx.reshape(BTGT * TOPN),
    )

    gcol = (cidx.reshape(BTGT * TOPN, 1) * CHUNK +
            jnp.arange(CHUNK, dtype=jnp.int32).reshape(1, CHUNK))

    loss2, nc2 = pl.pallas_call(
        _loss_body,
        out_shape=[
            jax.ShapeDtypeStruct((1, 1), f32),
            jax.ShapeDtypeStruct((1, 1), jnp.int32),
        ],
    )(
        cand,
        cand0,
        clab,
        gcol,
        logz,
        source_labels.astype(f32).reshape(BSRC, 1),
        target_labels.astype(jnp.int32).reshape(BTGT, 1),
    )
    return (loss2[0, 0], nc2[0, 0])


# speedup vs baseline: 5.5197x; 1.0649x over previous
import functools

import jax
import jax.numpy as jnp
from jax import lax
from jax.experimental import pallas as pl
from jax.experimental.pallas import tpu as pltpu
from jax.experimental.pallas import tpu_sc as plsc

DIM = 512
KQ = 48000
TEMP = 0.007
TOPN = 32
RK = 4
BSRC = 64
BTGT = 256
NB = 15
LBLK = KQ // NB
CHUNK = 128
CPB = LBLK // CHUNK
NCH = KQ // CHUNK
CW = TOPN * CHUNK
BIG = 1 << 30


def _sim_body(feats_ref, tf0_ref, q_ref, sim_ref, sim0_ref, logz_ref,
              fidx_ref, cidx_ref, tn_ref, cm_ref, m_ref, s_ref):
    k = pl.program_id(0)

    @pl.when(k == 0)
    def _init():
        t = feats_ref[BSRC:, :]
        t0 = tf0_ref[...]
        tn_ref[0:BTGT, :] = t / (jnp.sqrt(jnp.sum(t * t, axis=1, keepdims=True)) + 1e-12)
        tn_ref[BTGT:, :] = t0 / (jnp.sqrt(jnp.sum(t0 * t0, axis=1, keepdims=True)) + 1e-12)
        m_ref[...] = jnp.full((BTGT, 1), -jnp.inf, jnp.float32)
        s_ref[...] = jnp.zeros((BTGT, 1), jnp.float32)

    qblk = q_ref[...]
    src = feats_ref[0:BSRC, :]
    head = jnp.where(k == 0, src, qblk[0:BSRC, :])
    q_eff = jnp.concatenate([head, qblk[BSRC:, :]], axis=0)
    qn = q_eff / (jnp.sqrt(jnp.sum(q_eff * q_eff, axis=1, keepdims=True)) + 1e-12)
    s_all = lax.dot_general(tn_ref[...], qn, (((1,), (1,)), ((), ())),
                            preferred_element_type=jnp.float32)
    sim_b = s_all[0:BTGT, :]
    sim0_b = s_all[BTGT:, :]
    sim_ref[...] = sim_b
    sim0_ref[...] = sim0_b
    cm_ref[pl.ds(k, 1)] = jnp.max(
        sim_b.reshape(BTGT, CPB, CHUNK), axis=2).reshape(1, BTGT, CPB)

    logits = sim_b / TEMP
    bm = jnp.max(logits, axis=1, keepdims=True)
    new_m = jnp.maximum(m_ref[...], bm)
    s_ref[...] = s_ref[...] * jnp.exp(m_ref[...] - new_m) + \
        jnp.sum(jnp.exp(logits - new_m), axis=1, keepdims=True)
    m_ref[...] = new_m

    @pl.when(k == NB - 1)
    def _fin():
        logz_ref[...] = m_ref[...] + jnp.log(s_ref[...])
        cm0 = jnp.concatenate([cm_ref[kk] for kk in range(NB)], axis=1)
        col = lax.broadcasted_iota(jnp.int32, (BTGT, NCH), 1)
        row = lax.broadcasted_iota(jnp.int32, (BTGT, 1), 0)
        ncol = lax.broadcasted_iota(jnp.int32, (BTGT, TOPN), 1)

        def step(n, carry):
            cm, acc_f, acc_c = carry
            mx = jnp.max(cm, axis=1, keepdims=True)
            idx = jnp.min(jnp.where(cm == mx, col, BIG), axis=1, keepdims=True)
            cm = jnp.where(col == idx, -jnp.inf, cm)
            acc_c = jnp.where(ncol == n, idx, acc_c)
            acc_f = jnp.where(ncol == n, row * NCH + idx, acc_f)
            return cm, acc_f, acc_c

        zero = jnp.zeros((BTGT, TOPN), jnp.int32)
        _, acc_f, acc_c = lax.fori_loop(0, TOPN, step, (cm0, zero, zero))
        fidx_ref[...] = acc_f
        cidx_ref[...] = acc_c


def _gather_sc(sim2d, sim02d, lab2d, fidx, cidx):
    info = plsc.get_sparse_core_info()
    nw = info.num_cores * info.num_subcores
    nr = BTGT * TOPN
    bw = nr // nw
    half = bw // 2
    mesh = plsc.VectorSubcoreMesh(core_axis_name="c", subcore_axis_name="s")

    @functools.partial(
        pl.kernel,
        out_type=[
            jax.ShapeDtypeStruct((nr, CHUNK), jnp.float32),
            jax.ShapeDtypeStruct((nr, CHUNK), jnp.float32),
            jax.ShapeDtypeStruct((nr, CHUNK), jnp.int32),
        ],
        mesh=mesh,
        scratch_types=[
            pltpu.VMEM((half,), jnp.int32),
            pltpu.VMEM((half,), jnp.int32),
            pltpu.VMEM((half,), jnp.int32),
            pltpu.VMEM((half,), jnp.int32),
            pltpu.VMEM((half, CHUNK), jnp.float32),
            pltpu.VMEM((half, CHUNK), jnp.float32),
            pltpu.VMEM((half, CHUNK), jnp.float32),
            pltpu.VMEM((half, CHUNK), jnp.float32),
            pltpu.VMEM((half, CHUNK), jnp.int32),
            pltpu.VMEM((half, CHUNK), jnp.int32),
            pltpu.SemaphoreType.DMA,
            pltpu.SemaphoreType.DMA,
        ],
    )
    def gather_kernel(sim_hbm, sim0_hbm, lab_hbm, fidx_hbm, cidx_hbm,
                      cand_hbm, cand0_hbm, clab_hbm,
                      idx_a, idx_b, cidx_a, cidx_b,
                      buf_sa, buf_sb, buf_0a, buf_0b, buf_la, buf_lb,
                      gsem, wsem):
        wid = lax.axis_index("s") * info.num_cores + lax.axis_index("c")
        base = wid * bw
        pltpu.sync_copy(fidx_hbm.at[pl.ds(base, half)], idx_a)
        pltpu.sync_copy(fidx_hbm.at[pl.ds(base + half, half)], idx_b)
        pltpu.sync_copy(cidx_hbm.at[pl.ds(base, half)], cidx_a)
        pltpu.sync_copy(cidx_hbm.at[pl.ds(base + half, half)], cidx_b)
        gathers = [
            (sim_hbm, idx_a, buf_sa, cand_hbm, base),
            (sim_hbm, idx_b, buf_sb, cand_hbm, base + half),
            (sim0_hbm, idx_a, buf_0a, cand0_hbm, base),
            (sim0_hbm, idx_b, buf_0b, cand0_hbm, base + half),
            (lab_hbm, cidx_a, buf_la, clab_hbm, base),
            (lab_hbm, cidx_b, buf_lb, clab_hbm, base + half),
        ]
        copies = [pltpu.async_copy(tbl.at[idx], buf, gsem)
                  for tbl, idx, buf, _, _ in gathers]
        writes = []
        for cp, (_, _, buf, out, off) in zip(copies, gathers):
            cp.wait()
            writes.append(pltpu.async_copy(buf, out.at[pl.ds(off, half)], wsem))
        for wr in writes:
            wr.wait()

    return gather_kernel(sim2d, sim02d, lab2d, fidx, cidx)


def _loss_body(cand_ref, cand0_ref, clab_ref, gcol_ref, logz_ref, srcl_ref,
               tgt_ref, loss_ref, nc_ref):
    cand = cand_ref[...]
    col = lax.broadcasted_iota(jnp.int32, (BTGT, CW), 1)

    def mask_step(_, v):
        m = jnp.max(v, axis=1, keepdims=True)
        return jnp.where(v == m, -jnp.inf, v)

    v = lax.fori_loop(0, TOPN - 1, mask_step, cand)
    theta = jnp.max(v, axis=1, keepdims=True)
    rank = jnp.where(cand >= theta, cand + cand0_ref[...], -jnp.inf)

    logz = logz_ref[...]
    vsum = jnp.zeros((BTGT, 1), jnp.float32)
    g0 = jnp.zeros((BTGT, 1), jnp.int32)
    lab_q = jnp.zeros((BTGT, 1), jnp.int32)
    for n in range(RK):
        m = jnp.max(rank, axis=1, keepdims=True)
        eq = rank == m
        pos = jnp.min(jnp.where(eq, col, BIG), axis=1, keepdims=True)
        sel = col == pos
        val = jnp.sum(jnp.where(sel, cand, 0.0), axis=1, keepdims=True)
        vsum = vsum + (val / TEMP - logz)
        if n == 0:
            g0 = jnp.sum(jnp.where(sel, gcol_ref[...], 0), axis=1, keepdims=True)
            lab_q = jnp.sum(jnp.where(sel, clab_ref[...], 0), axis=1, keepdims=True)
        rank = jnp.where(sel, -jnp.inf, rank)

    loss_ref[...] = -(jnp.sum(vsum, axis=0, keepdims=True) / (BTGT * RK))

    iota64 = lax.broadcasted_iota(jnp.int32, (BTGT, BSRC), 1)
    onehot = jnp.where(iota64 == g0, 1.0, 0.0)
    src_val = lax.dot_general(onehot, srcl_ref[...], (((1,), (0,)), ((), ())),
                              preferred_element_type=jnp.float32)
    pred = jnp.where(g0 < BSRC, src_val.astype(jnp.int32), lab_q)
    nc_ref[...] = jnp.sum(jnp.where(pred == tgt_ref[...], 1, 0),
                          axis=0, keepdims=True)


def kernel(features, target_fearures_0, source_labels, target_labels, queue,
           queue_labels):
    f32 = jnp.float32
    sim, sim0, logz, fidx, cidx = pl.pallas_call(
        _sim_body,
        grid=(NB,),
        in_specs=[
            pl.BlockSpec((BSRC + BTGT, DIM), lambda k: (0, 0)),
            pl.BlockSpec((BTGT, DIM), lambda k: (0, 0)),
            pl.BlockSpec((LBLK, DIM), lambda k: (k, 0)),
        ],
        out_specs=[
            pl.BlockSpec((BTGT, LBLK), lambda k: (0, k)),
            pl.BlockSpec((BTGT, LBLK), lambda k: (0, k)),
            pl.BlockSpec((BTGT, 1), lambda k: (0, 0)),
            pl.BlockSpec((BTGT, TOPN), lambda k: (0, 0)),
            pl.BlockSpec((BTGT, TOPN), lambda k: (0, 0)),
        ],
        out_shape=[
            jax.ShapeDtypeStruct((BTGT, KQ), f32),
            jax.ShapeDtypeStruct((BTGT, KQ), f32),
            jax.ShapeDtypeStruct((BTGT, 1), f32),
            jax.ShapeDtypeStruct((BTGT, TOPN), jnp.int32),
            jax.ShapeDtypeStruct((BTGT, TOPN), jnp.int32),
        ],
        scratch_shapes=[
            pltpu.VMEM((BTGT + BTGT, DIM), f32),
            pltpu.VMEM((NB, BTGT, CPB), f32),
            pltpu.VMEM((BTGT, 1), f32),
            pltpu.VMEM((BTGT, 1), f32),
        ],
    )(features, target_fearures_0, queue)

    cand, cand0, clab = _gather_sc(
        sim.reshape(BTGT * NCH, CHUNK),
        sim0.reshape(BTGT * NCH, CHUNK),
        queue_labels.reshape(NCH, CHUNK),
        fidx.reshape(BTGT * TOPN),
        cidx.reshape(BTGT * TOPN),
    )

    gcol = (cidx.reshape(BTGT, TOPN, 1) * CHUNK +
            jnp.arange(CHUNK, dtype=jnp.int32).reshape(1, 1, CHUNK)
            ).reshape(BTGT, CW)

    loss2, nc2 = pl.pallas_call(
        _loss_body,
        out_shape=[
            jax.ShapeDtypeStruct((1, 1), f32),
            jax.ShapeDtypeStruct((1, 1), jnp.int32),
        ],
    )(
        cand.reshape(BTGT, CW),
        cand0.reshape(BTGT, CW),
        clab.reshape(BTGT, CW),
        gcol,
        logz,
        source_labels.astype(f32).reshape(BSRC, 1),
        target_labels.astype(jnp.int32).reshape(BTGT, 1),
    )
    return (loss2[0, 0], nc2[0, 0])
